# Initial kernel scaffold; baseline (speedup 1.0000x reference)
import jax
import jax.numpy as jnp
from jax.experimental import pallas as pl


def kernel(x, adj, W1, b1, W2, b2, W3, b3, Wl, bl):
    raise NotImplementedError("write your pallas kernel here")


# speedup vs baseline: 1.0050x; 1.0050x over previous
import functools
import jax
import jax.numpy as jnp
from jax.experimental import pallas as pl
from jax.experimental.pallas import tpu as pltpu

_N = 10000
_BI = 400


def _xw_body(x_ref, w_ref, o_ref):
    o_ref[...] = jnp.dot(x_ref[...], w_ref[...],
                         preferred_element_type=jnp.float32)


def _layer_body(adj_ref, z_ref, b_ref, wn_ref, o_ref):
    h = jnp.dot(adj_ref[...], z_ref[...], preferred_element_type=jnp.float32)
    h = jnp.maximum(h + b_ref[...], 0.0)
    o_ref[...] = jnp.dot(h, wn_ref[...], preferred_element_type=jnp.float32)


def _final_body(adj_ref, z_ref, b_ref, wl_ref, bl_ref, o_ref):
    h = jnp.dot(adj_ref[...], z_ref[...], preferred_element_type=jnp.float32)
    h = jnp.maximum(h + b_ref[...], 0.0)
    logits = jnp.dot(h, wl_ref[...],
                     preferred_element_type=jnp.float32) + bl_ref[...]
    m = jnp.max(logits, axis=1, keepdims=True)
    s = logits - m
    lse = jnp.log(jnp.sum(jnp.exp(s), axis=1, keepdims=True))
    o_ref[...] = s - lse


def _small_matmul(x, w):
    n, k = x.shape
    _, m = w.shape
    return pl.pallas_call(
        _xw_body,
        out_shape=jax.ShapeDtypeStruct((n, m), jnp.float32),
        in_specs=[
            pl.BlockSpec((n, k), lambda: (0, 0)),
            pl.BlockSpec((k, m), lambda: (0, 0)),
        ],
        out_specs=pl.BlockSpec((n, m), lambda: (0, 0)),
    )(x, w)


def _spmm_layer(adj, z, b, wn):
    n = adj.shape[0]
    grid = (n // _BI,)
    return pl.pallas_call(
        _layer_body,
        grid=grid,
        out_shape=jax.ShapeDtypeStruct((n, wn.shape[1]), jnp.float32),
        in_specs=[
            pl.BlockSpec((_BI, n), lambda i: (i, 0)),
            pl.BlockSpec((n, z.shape[1]), lambda i: (0, 0)),
            pl.BlockSpec((1, b.shape[1]), lambda i: (0, 0)),
            pl.BlockSpec(wn.shape, lambda i: (0, 0)),
        ],
        out_specs=pl.BlockSpec((_BI, wn.shape[1]), lambda i: (i, 0)),
        compiler_params=pltpu.CompilerParams(
            dimension_semantics=("arbitrary",),
        ),
    )(adj, z, b, wn)


def _spmm_final(adj, z, b, wl, bl):
    n = adj.shape[0]
    nclass = wl.shape[1]
    grid = (n // _BI,)
    return pl.pallas_call(
        _final_body,
        grid=grid,
        out_shape=jax.ShapeDtypeStruct((n, nclass), jnp.float32),
        in_specs=[
            pl.BlockSpec((_BI, n), lambda i: (i, 0)),
            pl.BlockSpec((n, z.shape[1]), lambda i: (0, 0)),
            pl.BlockSpec((1, b.shape[1]), lambda i: (0, 0)),
            pl.BlockSpec(wl.shape, lambda i: (0, 0)),
            pl.BlockSpec((1, nclass), lambda i: (0, 0)),
        ],
        out_specs=pl.BlockSpec((_BI, nclass), lambda i: (i, 0)),
        compiler_params=pltpu.CompilerParams(
            dimension_semantics=("arbitrary",),
        ),
    )(adj, z, b, wl, bl)


def kernel(x, adj, W1, b1, W2, b2, W3, b3, Wl, bl):
    b1 = b1.reshape(1, -1)
    b2 = b2.reshape(1, -1)
    b3 = b3.reshape(1, -1)
    bl = bl.reshape(1, -1)
    z0 = _small_matmul(x, W1)
    z1 = _spmm_layer(adj, z0, b1, W2)
    z2 = _spmm_layer(adj, z1, b2, W3)
    out = _spmm_final(adj, z2, b3, Wl, bl)
    return out


# speedup vs baseline: 1.0800x; 1.0746x over previous
import functools
import jax
import jax.numpy as jnp
from jax.experimental import pallas as pl
from jax.experimental.pallas import tpu as pltpu

_N = 10000
_BI = 400


def _xw_body(x_ref, w_ref, o_ref):
    o_ref[...] = jnp.dot(x_ref[...], w_ref[...],
                         preferred_element_type=jnp.float32)


def _layer1_body(adj_ref, z_ref, b_ref, wn_ref, o_ref, adj16_ref):
    a = adj_ref[...]
    adj16_ref[...] = a.astype(jnp.bfloat16)
    h = jnp.dot(a, z_ref[...], preferred_element_type=jnp.float32)
    h = jnp.maximum(h + b_ref[...], 0.0)
    o_ref[...] = jnp.dot(h, wn_ref[...], preferred_element_type=jnp.float32)


def _layer_body(adj_ref, z_ref, b_ref, wn_ref, o_ref):
    h = jnp.dot(adj_ref[...], z_ref[...].astype(jnp.bfloat16),
                preferred_element_type=jnp.float32)
    h = jnp.maximum(h + b_ref[...], 0.0)
    o_ref[...] = jnp.dot(h, wn_ref[...], preferred_element_type=jnp.float32)


def _final_body(adj_ref, z_ref, b_ref, wl_ref, bl_ref, o_ref):
    h = jnp.dot(adj_ref[...], z_ref[...].astype(jnp.bfloat16),
                preferred_element_type=jnp.float32)
    h = jnp.maximum(h + b_ref[...], 0.0)
    logits = jnp.dot(h, wl_ref[...],
                     preferred_element_type=jnp.float32) + bl_ref[...]
    m = jnp.max(logits, axis=1, keepdims=True)
    s = logits - m
    lse = jnp.log(jnp.sum(jnp.exp(s), axis=1, keepdims=True))
    o_ref[...] = s - lse


def _small_matmul(x, w):
    n, k = x.shape
    _, m = w.shape
    return pl.pallas_call(
        _xw_body,
        out_shape=jax.ShapeDtypeStruct((n, m), jnp.float32),
        in_specs=[
            pl.BlockSpec((n, k), lambda: (0, 0)),
            pl.BlockSpec((k, m), lambda: (0, 0)),
        ],
        out_specs=pl.BlockSpec((n, m), lambda: (0, 0)),
    )(x, w)


def _spmm_layer1(adj, z, b, wn):
    n = adj.shape[0]
    grid = (n // _BI,)
    return pl.pallas_call(
        _layer1_body,
        grid=grid,
        out_shape=[
            jax.ShapeDtypeStruct((n, wn.shape[1]), jnp.float32),
            jax.ShapeDtypeStruct((n, n), jnp.bfloat16),
        ],
        in_specs=[
            pl.BlockSpec((_BI, n), lambda i: (i, 0)),
            pl.BlockSpec((n, z.shape[1]), lambda i: (0, 0)),
            pl.BlockSpec((1, b.shape[1]), lambda i: (0, 0)),
            pl.BlockSpec(wn.shape, lambda i: (0, 0)),
        ],
        out_specs=[
            pl.BlockSpec((_BI, wn.shape[1]), lambda i: (i, 0)),
            pl.BlockSpec((_BI, n), lambda i: (i, 0)),
        ],
        compiler_params=pltpu.CompilerParams(
            dimension_semantics=("arbitrary",),
        ),
    )(adj, z, b, wn)


def _spmm_layer(adj16, z, b, wn):
    n = adj16.shape[0]
    grid = (n // _BI,)
    return pl.pallas_call(
        _layer_body,
        grid=grid,
        out_shape=jax.ShapeDtypeStruct((n, wn.shape[1]), jnp.float32),
        in_specs=[
            pl.BlockSpec((_BI, n), lambda i: (i, 0)),
            pl.BlockSpec((n, z.shape[1]), lambda i: (0, 0)),
            pl.BlockSpec((1, b.shape[1]), lambda i: (0, 0)),
            pl.BlockSpec(wn.shape, lambda i: (0, 0)),
        ],
        out_specs=pl.BlockSpec((_BI, wn.shape[1]), lambda i: (i, 0)),
        compiler_params=pltpu.CompilerParams(
            dimension_semantics=("arbitrary",),
        ),
    )(adj16, z, b, wn)


def _spmm_final(adj, z, b, wl, bl):
    n = adj.shape[0]
    nclass = wl.shape[1]
    grid = (n // _BI,)
    return pl.pallas_call(
        _final_body,
        grid=grid,
        out_shape=jax.ShapeDtypeStruct((n, nclass), jnp.float32),
        in_specs=[
            pl.BlockSpec((_BI, n), lambda i: (i, 0)),
            pl.BlockSpec((n, z.shape[1]), lambda i: (0, 0)),
            pl.BlockSpec((1, b.shape[1]), lambda i: (0, 0)),
            pl.BlockSpec(wl.shape, lambda i: (0, 0)),
            pl.BlockSpec((1, nclass), lambda i: (0, 0)),
        ],
        out_specs=pl.BlockSpec((_BI, nclass), lambda i: (i, 0)),
        compiler_params=pltpu.CompilerParams(
            dimension_semantics=("arbitrary",),
        ),
    )(adj, z, b, wl, bl)


def kernel(x, adj, W1, b1, W2, b2, W3, b3, Wl, bl):
    b1 = b1.reshape(1, -1)
    b2 = b2.reshape(1, -1)
    b3 = b3.reshape(1, -1)
    bl = bl.reshape(1, -1)
    z0 = _small_matmul(x, W1)
    z1, adj16 = _spmm_layer1(adj, z0, b1, W2)
    z2 = _spmm_layer(adj16, z1, b2, W3)
    out = _spmm_final(adj16, z2, b3, Wl, bl)
    return out
